# Initial kernel scaffold; baseline (speedup 1.0000x reference)
import jax
import jax.numpy as jnp
from jax.experimental import pallas as pl


def kernel(x, one_hot, codebook_shared, codebook_task):
    raise NotImplementedError("write your pallas kernel here")


# speedup vs baseline: 1.5533x; 1.5533x over previous
import functools

import jax
import jax.numpy as jnp
from jax.experimental import pallas as pl
from jax.experimental.pallas import tpu as pltpu

N_E_S = 512
N_E_T = 128
D_S = 1024
D_T = 4
N_TASKS = 4
TILE_N = 512


def _vq_kernel(x_ref, cbs_ref, cbt_ref, oh_ref,
               out_ref, idxs_ref, idxt_ref, loss_ref,
               ens_ref, ent_ref, *, nb, nt, inv_count):
    b = pl.program_id(0)
    n = pl.program_id(1)
    first = jnp.logical_and(b == 0, n == 0)
    last = jnp.logical_and(b == nb - 1, n == nt - 1)

    @pl.when(first)
    def _():
        cbs = cbs_ref[...]
        ens_ref[...] = jnp.sum(cbs * cbs, axis=1, keepdims=True)
        cbt = cbt_ref[...]
        ent_ref[...] = jnp.sum(cbt * cbt, axis=1, keepdims=True)
        loss_ref[...] = jnp.zeros_like(loss_ref)

    zs = x_ref[0, 0:D_S, :]
    zt = x_ref[0, D_S:D_S + D_T, :]

    cbs = cbs_ref[...]
    m_s = jax.lax.dot_general(
        cbs, zs, (((1,), (0,)), ((), ())),
        precision=jax.lax.Precision.DEFAULT,
        preferred_element_type=jnp.float32)
    a_s = jnp.sum(zs * zs, axis=0, keepdims=True)
    d_s = (a_s - 2.0 * m_s) + ens_ref[...]
    minv_s = jnp.min(d_s, axis=0, keepdims=True)
    row_s = jax.lax.broadcasted_iota(jnp.int32, (N_E_S, TILE_N), 0)
    idx_s = jnp.min(jnp.where(d_s == minv_s, row_s, N_E_S),
                    axis=0, keepdims=True)
    sel_s = (row_s == idx_s).astype(jnp.float32)
    zq_s = jax.lax.dot_general(
        cbs, sel_s, (((0,), (0,)), ((), ())),
        precision=jax.lax.Precision.HIGHEST,
        preferred_element_type=jnp.float32)

    ohb = oh_ref[pl.ds(b, 1), :]
    tvec = jax.lax.broadcasted_iota(jnp.int32, (1, N_TASKS), 1).astype(jnp.float32)
    tb = jnp.sum(ohb * tvec, axis=1, keepdims=True).astype(jnp.int32)

    cbt = cbt_ref[...]
    m_t = jax.lax.dot_general(
        cbt, zt, (((1,), (0,)), ((), ())),
        precision=jax.lax.Precision.DEFAULT,
        preferred_element_type=jnp.float32)
    a_t = jnp.sum(zt * zt, axis=0, keepdims=True)
    d_t = (a_t - 2.0 * m_t) + ent_ref[...]
    rcol = jax.lax.broadcasted_iota(jnp.int32, (N_TASKS * N_E_T, 1), 0)
    inb = (rcol // N_E_T) == tb
    d_t = jnp.where(inb, d_t, jnp.inf)
    minv_t = jnp.min(d_t, axis=0, keepdims=True)
    row_t = jax.lax.broadcasted_iota(jnp.int32, (N_TASKS * N_E_T, TILE_N), 0)
    gidx_t = jnp.min(jnp.where(d_t == minv_t, row_t, N_TASKS * N_E_T),
                     axis=0, keepdims=True)
    sel_t = (row_t == gidx_t).astype(jnp.float32)
    zq_t = jax.lax.dot_general(
        cbt, sel_t, (((0,), (0,)), ((), ())),
        precision=jax.lax.Precision.HIGHEST,
        preferred_element_type=jnp.float32)

    idxs_ref[...] = idx_s.reshape(1, 1, TILE_N)
    idxt_ref[...] = (gidx_t - tb * N_E_T).reshape(1, 1, TILE_N)
    out_ref[0, 0:D_S, :] = zs + (zq_s - zs)
    out_ref[0, D_S:D_S + D_T, :] = zt + (zq_t - zt)

    df_s = zs - zq_s
    df_t = zt - zq_t
    part = (jnp.sum(df_s * df_s, axis=(0, 1), keepdims=True)
            + jnp.sum(df_t * df_t, axis=(0, 1), keepdims=True))
    acc = loss_ref[...] + part
    loss_ref[...] = jnp.where(last, acc * (1.25 * inv_count), acc)


@jax.jit
def kernel(x, one_hot, codebook_shared, codebook_task):
    B, C, D, H, W = x.shape
    N = D * H * W
    xr = x.reshape(B, C, N)
    cbt = codebook_task.reshape(N_TASKS * N_E_T, D_T)
    nt = N // TILE_N

    grid = (B, nt)
    kfn = functools.partial(_vq_kernel, nb=B, nt=nt,
                            inv_count=1.0 / float(B * N * C))
    out_q, idx_s, idx_t, loss = pl.pallas_call(
        kfn,
        grid=grid,
        in_specs=[
            pl.BlockSpec((1, C, TILE_N), lambda b, n: (b, 0, n)),
            pl.BlockSpec((N_E_S, D_S), lambda b, n: (0, 0)),
            pl.BlockSpec((N_TASKS * N_E_T, D_T), lambda b, n: (0, 0)),
            pl.BlockSpec((B, N_TASKS), lambda b, n: (0, 0)),
        ],
        out_specs=[
            pl.BlockSpec((1, C, TILE_N), lambda b, n: (b, 0, n)),
            pl.BlockSpec((1, 1, TILE_N), lambda b, n: (b, 0, n)),
            pl.BlockSpec((1, 1, TILE_N), lambda b, n: (b, 0, n)),
            pl.BlockSpec((1, 1), lambda b, n: (0, 0)),
        ],
        out_shape=[
            jax.ShapeDtypeStruct((B, C, N), jnp.float32),
            jax.ShapeDtypeStruct((B, 1, N), jnp.int32),
            jax.ShapeDtypeStruct((B, 1, N), jnp.int32),
            jax.ShapeDtypeStruct((1, 1), jnp.float32),
        ],
        scratch_shapes=[
            pltpu.VMEM((N_E_S, 1), jnp.float32),
            pltpu.VMEM((N_TASKS * N_E_T, 1), jnp.float32),
        ],
    )(xr, codebook_shared, cbt, one_hot)

    zq_fold = out_q.reshape(B, C, D, H, W)
    return (zq_fold, loss.reshape(()), idx_s.reshape(B, N), idx_t.reshape(B, N))


# speedup vs baseline: 2.1153x; 1.3618x over previous
import functools

import jax
import jax.numpy as jnp
from jax.experimental import pallas as pl
from jax.experimental.pallas import tpu as pltpu

N_E_S = 512
N_E_T = 128
D_S = 1024
D_T = 4
N_TASKS = 4
TILE_N = 512


def _vq_kernel(x_ref, cbs_ref, cbsh_ref, cbsm_ref, cbt_ref, cbth_ref,
               cbtm_ref, oh_ref,
               out_ref, idxs_ref, idxt_ref, loss_ref,
               ens_ref, ent_ref, *, nb, nt, inv_count):
    b = pl.program_id(0)
    n = pl.program_id(1)
    first = jnp.logical_and(b == 0, n == 0)
    last = jnp.logical_and(b == nb - 1, n == nt - 1)

    @pl.when(first)
    def _():
        cbs = cbs_ref[...]
        ens_ref[...] = jnp.sum(cbs * cbs, axis=1, keepdims=True)
        cbt = cbt_ref[...]
        ent_ref[...] = jnp.sum(cbt * cbt, axis=1, keepdims=True)
        loss_ref[...] = jnp.zeros_like(loss_ref)

    zs = x_ref[0, 0:D_S, :]
    zt = x_ref[0, D_S:D_S + D_T, :]

    m_s = jax.lax.dot_general(
        cbs_ref[...], zs, (((1,), (0,)), ((), ())),
        precision=jax.lax.Precision.DEFAULT,
        preferred_element_type=jnp.float32)
    a_s = jnp.sum(zs * zs, axis=0, keepdims=True)
    d_s = (a_s - 2.0 * m_s) + ens_ref[...]
    minv_s = jnp.min(d_s, axis=0, keepdims=True)
    row_s = jax.lax.broadcasted_iota(jnp.int32, (N_E_S, TILE_N), 0)
    idx_s = jnp.min(jnp.where(d_s == minv_s, row_s, N_E_S),
                    axis=0, keepdims=True)
    sel_s = (row_s == idx_s).astype(jnp.bfloat16)
    zq_s = (jax.lax.dot_general(
                cbsh_ref[...], sel_s, (((1,), (0,)), ((), ())),
                preferred_element_type=jnp.float32)
            + 0.0000152587890625 * jax.lax.dot_general(
                cbsm_ref[...], sel_s, (((1,), (0,)), ((), ())),
                preferred_element_type=jnp.float32))

    ohb = oh_ref[pl.ds(b, 1), :]
    tvec = jax.lax.broadcasted_iota(jnp.int32, (1, N_TASKS), 1).astype(jnp.float32)
    tb = jnp.sum(ohb * tvec, axis=1, keepdims=True).astype(jnp.int32)

    m_t = jax.lax.dot_general(
        cbt_ref[...], zt, (((1,), (0,)), ((), ())),
        precision=jax.lax.Precision.DEFAULT,
        preferred_element_type=jnp.float32)
    a_t = jnp.sum(zt * zt, axis=0, keepdims=True)
    d_t = (a_t - 2.0 * m_t) + ent_ref[...]
    rcol = jax.lax.broadcasted_iota(jnp.int32, (N_TASKS * N_E_T, 1), 0)
    inb = (rcol // N_E_T) == tb
    d_t = jnp.where(inb, d_t, jnp.inf)
    minv_t = jnp.min(d_t, axis=0, keepdims=True)
    row_t = jax.lax.broadcasted_iota(jnp.int32, (N_TASKS * N_E_T, TILE_N), 0)
    gidx_t = jnp.min(jnp.where(d_t == minv_t, row_t, N_TASKS * N_E_T),
                     axis=0, keepdims=True)
    sel_t = (row_t == gidx_t).astype(jnp.bfloat16)
    zq_t = (jax.lax.dot_general(
                cbth_ref[...], sel_t, (((1,), (0,)), ((), ())),
                preferred_element_type=jnp.float32)
            + 0.0000152587890625 * jax.lax.dot_general(
                cbtm_ref[...], sel_t, (((1,), (0,)), ((), ())),
                preferred_element_type=jnp.float32))

    idxs_ref[...] = idx_s.reshape(1, 1, TILE_N)
    idxt_ref[...] = (gidx_t - tb * N_E_T).reshape(1, 1, TILE_N)
    out_ref[0, 0:D_S, :] = zq_s
    out_ref[0, D_S:D_S + D_T, :] = zq_t

    part = (jnp.sum(minv_s, axis=(0, 1), keepdims=True)
            + jnp.sum(minv_t, axis=(0, 1), keepdims=True))
    acc = loss_ref[...] + part
    loss_ref[...] = jnp.where(last, acc * (1.25 * inv_count), acc)


@jax.jit
def kernel(x, one_hot, codebook_shared, codebook_task):
    B, C, D, H, W = x.shape
    N = D * H * W
    xr = x.reshape(B, C, N)
    cbt = codebook_task.reshape(N_TASKS * N_E_T, D_T)
    nt = N // TILE_N

    cbs_t = codebook_shared.T
    cbsh = cbs_t.astype(jnp.bfloat16)
    cbsm = ((cbs_t - cbsh.astype(jnp.float32)) * 65536.0).astype(jnp.bfloat16)
    cbt_t = cbt.T
    cbth = cbt_t.astype(jnp.bfloat16)
    cbtm = ((cbt_t - cbth.astype(jnp.float32)) * 65536.0).astype(jnp.bfloat16)

    grid = (B, nt)
    kfn = functools.partial(_vq_kernel, nb=B, nt=nt,
                            inv_count=1.0 / float(B * N * C))
    out_q, idx_s, idx_t, loss = pl.pallas_call(
        kfn,
        grid=grid,
        in_specs=[
            pl.BlockSpec((1, C, TILE_N), lambda b, n: (b, 0, n)),
            pl.BlockSpec((N_E_S, D_S), lambda b, n: (0, 0)),
            pl.BlockSpec((D_S, N_E_S), lambda b, n: (0, 0)),
            pl.BlockSpec((D_S, N_E_S), lambda b, n: (0, 0)),
            pl.BlockSpec((N_TASKS * N_E_T, D_T), lambda b, n: (0, 0)),
            pl.BlockSpec((D_T, N_TASKS * N_E_T), lambda b, n: (0, 0)),
            pl.BlockSpec((D_T, N_TASKS * N_E_T), lambda b, n: (0, 0)),
            pl.BlockSpec((B, N_TASKS), lambda b, n: (0, 0)),
        ],
        out_specs=[
            pl.BlockSpec((1, C, TILE_N), lambda b, n: (b, 0, n)),
            pl.BlockSpec((1, 1, TILE_N), lambda b, n: (b, 0, n)),
            pl.BlockSpec((1, 1, TILE_N), lambda b, n: (b, 0, n)),
            pl.BlockSpec((1, 1), lambda b, n: (0, 0)),
        ],
        out_shape=[
            jax.ShapeDtypeStruct((B, C, N), jnp.float32),
            jax.ShapeDtypeStruct((B, 1, N), jnp.int32),
            jax.ShapeDtypeStruct((B, 1, N), jnp.int32),
            jax.ShapeDtypeStruct((1, 1), jnp.float32),
        ],
        scratch_shapes=[
            pltpu.VMEM((N_E_S, 1), jnp.float32),
            pltpu.VMEM((N_TASKS * N_E_T, 1), jnp.float32),
        ],
    )(xr, codebook_shared, cbsh, cbsm, cbt, cbth, cbtm, one_hot)

    zq_fold = out_q.reshape(B, C, D, H, W)
    return (zq_fold, loss.reshape(()), idx_s.reshape(B, N), idx_t.reshape(B, N))
